# baseline (speedup 1.0000x reference)
import functools

import jax
import jax.numpy as jnp
from jax import lax
from jax.experimental import pallas as pl
from jax.experimental.pallas import tpu as pltpu
from jax.experimental.pallas import tpu_sc as plsc

VOCAB = 32000
RES = 2048
R_OUT = 512
BATCH = 16
T = 32
ROWS = BATCH * T

NC = 2
NS = 16
NW = NC * NS
B_PER_W = ROWS // NW

V_TILE = 3200
N_VT = VOCAB // V_TILE
S0 = 24
S1 = T - S0
ABUF_N = 3
OBUF_N = 2


def _make_gather():
    mesh = plsc.VectorSubcoreMesh(core_axis_name="c", subcore_axis_name="s")

    @functools.partial(
        pl.kernel,
        mesh=mesh,
        out_type=jax.ShapeDtypeStruct((ROWS, RES), jnp.float32),
        scratch_types=[
            pltpu.VMEM((B_PER_W // 2,), jnp.int32),
            pltpu.VMEM((B_PER_W // 2,), jnp.int32),
            pltpu.VMEM((B_PER_W // 2, RES), jnp.float32),
            pltpu.VMEM((B_PER_W // 2, RES), jnp.float32),
            pltpu.SemaphoreType.DMA,
            pltpu.SemaphoreType.DMA,
        ],
    )
    def gather(table_hbm, idx_hbm, out_hbm, idx_a, idx_b, rows_a, rows_b,
               sem_a, sem_b):
        wid = lax.axis_index("s") * NC + lax.axis_index("c")
        half = B_PER_W // 2
        base = wid * B_PER_W
        pltpu.sync_copy(idx_hbm.at[pl.ds(base, half)], idx_a)
        pltpu.sync_copy(idx_hbm.at[pl.ds(base + half, half)], idx_b)
        ga = pltpu.async_copy(table_hbm.at[idx_a], rows_a, sem_a)
        gb = pltpu.async_copy(table_hbm.at[idx_b], rows_b, sem_b)
        ga.wait()
        pltpu.sync_copy(rows_a, out_hbm.at[pl.ds(base, half)])
        gb.wait()
        pltpu.sync_copy(rows_b, out_hbm.at[pl.ds(base + half, half)])

    return gather


_make_gather = functools.cache(_make_gather)


def _fused_body(u_ref, w_ref, a_ref, h0_ref, bw_ref, ab_ref, aw_ref,
                out_ref, hsc, zsc, abuf, obuf, asem, osem):
    def step(t, h):
        a = a_ref[:]
        u = u_ref[pl.ds(t * BATCH, BATCH), :]
        rec = lax.dot_general(h, w_ref[:], (((1,), (1,)), ((), ())),
                              preferred_element_type=jnp.float32)
        pre = jnp.clip(u + rec, -10.0, 10.0)
        hn = (1.0 - a) * h + a * jnp.tanh(pre)
        hsc[:, pl.ds(t, 1), :] = hn[:, None, :]
        return hn

    def z_block(toff, tlen):
        hs = hsc[:, pl.ds(toff, tlen), :]
        zv = lax.dot_general(hs, bw_ref[:], (((2,), (1,)), ((), ())),
                             preferred_element_type=jnp.float32)
        zsc[:, pl.ds(toff, tlen), :] = zv

    def a_copy(k):
        return pltpu.make_async_copy(
            aw_ref.at[pl.ds((k % N_VT) * V_TILE, V_TILE), :],
            abuf.at[k % ABUF_N],
            asem.at[k % ABUF_N])

    def o_copy(k, toff, tlen):
        return pltpu.make_async_copy(
            obuf.at[k % OBUF_N, :, pl.ds(0, tlen), :],
            out_ref.at[:, pl.ds(toff, tlen),
                       pl.ds((k % N_VT) * V_TILE, V_TILE)],
            osem.at[k % OBUF_N])

    def ro_tile(k, toff, tlen):
        zh = zsc[:, pl.ds(toff, tlen), :]
        A = abuf[k % ABUF_N]
        o = lax.dot_general(zh, A, (((2,), (1,)), ((), ())),
                            preferred_element_type=jnp.float32)
        bias = ab_ref[:, pl.ds((k % N_VT) * V_TILE, V_TILE)]
        obuf[k % OBUF_N, :, pl.ds(0, tlen), :] = o + bias[None, :, :]

    for k in range(ABUF_N):
        a_copy(k).start()

    h = jnp.broadcast_to(h0_ref[:], (BATCH, RES))
    h = lax.fori_loop(0, S0 // 2, lambda i, hh: step(2 * i + 1, step(2 * i, hh)), h)
    z_block(0, S0)

    bounds = [S0 + (i * S1) // N_VT for i in range(N_VT + 1)]
    for i in range(N_VT):
        for t in range(bounds[i], bounds[i + 1]):
            h = step(t, h)
        if i >= OBUF_N:
            o_copy(i - OBUF_N, 0, S0).wait()
        a_copy(i).wait()
        ro_tile(i, 0, S0)
        o_copy(i, 0, S0).start()
        if i + ABUF_N < 2 * N_VT:
            a_copy(i + ABUF_N).start()

    z_block(S0, S1)
    for i in range(N_VT, 2 * N_VT):
        if i - OBUF_N < N_VT:
            o_copy(i - OBUF_N, 0, S0).wait()
        else:
            o_copy(i - OBUF_N, S0, S1).wait()
        a_copy(i).wait()
        ro_tile(i, S0, S1)
        o_copy(i, S0, S1).start()
        if i + ABUF_N < 2 * N_VT:
            a_copy(i + ABUF_N).start()

    o_copy(2 * N_VT - 2, S0, S1).wait()
    o_copy(2 * N_VT - 1, S0, S1).wait()


def kernel(x, W_in, W_rec, a, B_w, A_w, A_b, h0):
    idx = x.astype(jnp.int32).T.reshape(ROWS)
    U = _make_gather()(W_in, idx)

    logits = pl.pallas_call(
        _fused_body,
        in_specs=[
            pl.BlockSpec(memory_space=pltpu.MemorySpace.VMEM),
            pl.BlockSpec(memory_space=pltpu.MemorySpace.VMEM),
            pl.BlockSpec(memory_space=pltpu.MemorySpace.VMEM),
            pl.BlockSpec(memory_space=pltpu.MemorySpace.VMEM),
            pl.BlockSpec(memory_space=pltpu.MemorySpace.VMEM),
            pl.BlockSpec(memory_space=pltpu.MemorySpace.VMEM),
            pl.BlockSpec(memory_space=pltpu.MemorySpace.HBM),
        ],
        out_specs=pl.BlockSpec(memory_space=pltpu.MemorySpace.HBM),
        out_shape=jax.ShapeDtypeStruct((BATCH, T, VOCAB), jnp.float32),
        scratch_shapes=[
            pltpu.VMEM((BATCH, T, RES), jnp.float32),
            pltpu.VMEM((BATCH, T, R_OUT), jnp.float32),
            pltpu.VMEM((ABUF_N, V_TILE, R_OUT), jnp.float32),
            pltpu.VMEM((OBUF_N, BATCH, S0, V_TILE), jnp.float32),
            pltpu.SemaphoreType.DMA((ABUF_N,)),
            pltpu.SemaphoreType.DMA((OBUF_N,)),
        ],
    )(U, W_rec, a.reshape(1, RES), h0.reshape(1, RES), B_w,
      A_b.reshape(1, VOCAB), A_w)

    return logits


# speedup vs baseline: 1.0013x; 1.0013x over previous
import functools

import jax
import jax.numpy as jnp
from jax import lax
from jax.experimental import pallas as pl
from jax.experimental.pallas import tpu as pltpu
from jax.experimental.pallas import tpu_sc as plsc

VOCAB = 32000
RES = 2048
R_OUT = 512
BATCH = 16
T = 32
ROWS = BATCH * T

NC = 2
NS = 16
NW = NC * NS
B_PER_W = ROWS // NW

V_TILE = 3200
N_VT = VOCAB // V_TILE
S0 = 24
S1 = T - S0
ABUF_N = 3
OBUF_N = 2


def _make_gather():
    mesh = plsc.VectorSubcoreMesh(core_axis_name="c", subcore_axis_name="s")

    @functools.partial(
        pl.kernel,
        mesh=mesh,
        out_type=jax.ShapeDtypeStruct((ROWS, RES), jnp.float32),
        scratch_types=[
            pltpu.VMEM((B_PER_W // 2,), jnp.int32),
            pltpu.VMEM((B_PER_W // 2,), jnp.int32),
            pltpu.VMEM((B_PER_W // 2, RES), jnp.float32),
            pltpu.VMEM((B_PER_W // 2, RES), jnp.float32),
            pltpu.SemaphoreType.DMA,
            pltpu.SemaphoreType.DMA,
        ],
    )
    def gather(table_hbm, idx_hbm, out_hbm, idx_a, idx_b, rows_a, rows_b,
               sem_a, sem_b):
        wid = lax.axis_index("s") * NC + lax.axis_index("c")
        half = B_PER_W // 2
        base = wid * B_PER_W
        pltpu.sync_copy(idx_hbm.at[pl.ds(base, half)], idx_a)
        pltpu.sync_copy(idx_hbm.at[pl.ds(base + half, half)], idx_b)
        ga = pltpu.async_copy(table_hbm.at[idx_a], rows_a, sem_a)
        gb = pltpu.async_copy(table_hbm.at[idx_b], rows_b, sem_b)
        ga.wait()
        pltpu.sync_copy(rows_a, out_hbm.at[pl.ds(base, half)])
        gb.wait()
        pltpu.sync_copy(rows_b, out_hbm.at[pl.ds(base + half, half)])

    return gather


_make_gather = functools.cache(_make_gather)


def _fused_body(u_ref, w_ref, a_ref, h0_ref, bw_ref, ab_ref, aw_ref,
                out_ref, hsc, zsc, abuf, obuf, asem, osem):
    def step(t, h):
        a = a_ref[:]
        u = u_ref[pl.ds(t * BATCH, BATCH), :]
        rec = lax.dot_general(h, w_ref[:], (((1,), (1,)), ((), ())),
                              preferred_element_type=jnp.float32)
        pre = jnp.clip(u + rec, -10.0, 10.0)
        hn = (1.0 - a) * h + a * jnp.tanh(pre)
        hsc[:, pl.ds(t, 1), :] = hn[:, None, :]
        return hn

    def z_block(toff, tlen):
        hs = hsc[:, pl.ds(toff, tlen), :]
        zv = lax.dot_general(hs, bw_ref[:], (((2,), (1,)), ((), ())),
                             preferred_element_type=jnp.float32)
        zsc[:, pl.ds(toff, tlen), :] = zv

    def a_copy(k):
        return pltpu.make_async_copy(
            aw_ref.at[pl.ds((k % N_VT) * V_TILE, V_TILE), :],
            abuf.at[k % ABUF_N],
            asem.at[k % ABUF_N])

    def o_copy(k, toff, tlen):
        return pltpu.make_async_copy(
            obuf.at[k % OBUF_N, :, pl.ds(0, tlen), :],
            out_ref.at[:, pl.ds(toff, tlen),
                       pl.ds((k % N_VT) * V_TILE, V_TILE)],
            osem.at[k % OBUF_N])

    def ro_tile(k, toff, tlen):
        zh = zsc[:, pl.ds(toff, tlen), :]
        A = abuf[k % ABUF_N]
        o = lax.dot_general(zh, A, (((2,), (1,)), ((), ())),
                            preferred_element_type=jnp.float32)
        bias = ab_ref[:, pl.ds((k % N_VT) * V_TILE, V_TILE)]
        obuf[k % OBUF_N, :, pl.ds(0, tlen), :] = o + bias[None, :, :]

    for k in range(ABUF_N):
        a_copy(k).start()

    h = jnp.broadcast_to(h0_ref[:], (BATCH, RES))
    h = lax.fori_loop(0, S0 // 2, lambda i, hh: step(2 * i + 1, step(2 * i, hh)), h)
    z_block(0, S0)

    bounds = [S0 + (i * S1) // N_VT for i in range(N_VT + 1)]
    for i in range(N_VT):
        if bounds[i + 1] > bounds[i]:
            h = lax.fori_loop(bounds[i], bounds[i + 1], step, h)
        if i >= OBUF_N:
            o_copy(i - OBUF_N, 0, S0).wait()
        a_copy(i).wait()
        ro_tile(i, 0, S0)
        o_copy(i, 0, S0).start()
        if i + ABUF_N < 2 * N_VT:
            a_copy(i + ABUF_N).start()

    z_block(S0, S1)
    for i in range(N_VT, 2 * N_VT):
        if i - OBUF_N < N_VT:
            o_copy(i - OBUF_N, 0, S0).wait()
        else:
            o_copy(i - OBUF_N, S0, S1).wait()
        a_copy(i).wait()
        ro_tile(i, S0, S1)
        o_copy(i, S0, S1).start()
        if i + ABUF_N < 2 * N_VT:
            a_copy(i + ABUF_N).start()

    o_copy(2 * N_VT - 2, S0, S1).wait()
    o_copy(2 * N_VT - 1, S0, S1).wait()


def kernel(x, W_in, W_rec, a, B_w, A_w, A_b, h0):
    idx = x.astype(jnp.int32).T.reshape(ROWS)
    U = _make_gather()(W_in, idx)

    logits = pl.pallas_call(
        _fused_body,
        in_specs=[
            pl.BlockSpec(memory_space=pltpu.MemorySpace.VMEM),
            pl.BlockSpec(memory_space=pltpu.MemorySpace.VMEM),
            pl.BlockSpec(memory_space=pltpu.MemorySpace.VMEM),
            pl.BlockSpec(memory_space=pltpu.MemorySpace.VMEM),
            pl.BlockSpec(memory_space=pltpu.MemorySpace.VMEM),
            pl.BlockSpec(memory_space=pltpu.MemorySpace.VMEM),
            pl.BlockSpec(memory_space=pltpu.MemorySpace.HBM),
        ],
        out_specs=pl.BlockSpec(memory_space=pltpu.MemorySpace.HBM),
        out_shape=jax.ShapeDtypeStruct((BATCH, T, VOCAB), jnp.float32),
        scratch_shapes=[
            pltpu.VMEM((BATCH, T, RES), jnp.float32),
            pltpu.VMEM((BATCH, T, R_OUT), jnp.float32),
            pltpu.VMEM((ABUF_N, V_TILE, R_OUT), jnp.float32),
            pltpu.VMEM((OBUF_N, BATCH, S0, V_TILE), jnp.float32),
            pltpu.SemaphoreType.DMA((ABUF_N,)),
            pltpu.SemaphoreType.DMA((OBUF_N,)),
        ],
    )(U, W_rec, a.reshape(1, RES), h0.reshape(1, RES), B_w,
      A_b.reshape(1, VOCAB), A_w)

    return logits
